# Initial kernel scaffold; baseline (speedup 1.0000x reference)
import functools

import jax
import jax.numpy as jnp
from jax import lax
from jax.experimental import pallas as pl
from jax.experimental.pallas import tpu as pltpu
from jax.experimental.pallas import tpu_sc as plsc

N = 10000
E = 160000
D = 256

NC = 2
NS = 16
NW = NC * NS


_BN = 2000
_BE = 2000


def _full(shape):
    return pl.BlockSpec(shape, lambda i: (0,) * len(shape))


def _rows(shape):
    return pl.BlockSpec(shape, lambda i: (i,) + (0,) * (len(shape) - 1))


def _node_proj_body(x_ref, w_ref, out_ref):
    d = jnp.dot(x_ref[...], w_ref[...], preferred_element_type=jnp.float32)
    out_ref[0] = d[:, :D]
    out_ref[1] = d[:, D:]


def _node_proj(x, w):
    return pl.pallas_call(
        _node_proj_body,
        grid=(N // _BN,),
        in_specs=[_rows((_BN, D)), _full((D, 2 * D))],
        out_specs=pl.BlockSpec((2, _BN, D), lambda i: (0, i, 0)),
        out_shape=jax.ShapeDtypeStruct((2, N, D), jnp.float32),
    )(x, w)


def _edge_mlp_body(gs_ref, gd_ref, e_ref, w1e_ref, b1_ref, w2_ref, b2_ref,
                   out_ref):
    t = (gs_ref[...] + gd_ref[...] + b1_ref[...]
         + jnp.dot(e_ref[...], w1e_ref[...],
                   preferred_element_type=jnp.float32))
    h = jnp.maximum(t, 0.0)
    out_ref[...] = (jnp.dot(h, w2_ref[...],
                            preferred_element_type=jnp.float32)
                    + b2_ref[...] + e_ref[...])


def _edge_mlp(gs, gd, e, w1e, b1, w2, b2):
    return pl.pallas_call(
        _edge_mlp_body,
        grid=(E // _BE,),
        in_specs=[_rows((_BE, D)), _rows((_BE, D)), _rows((_BE, D)),
                  _full((D, D)), _full((1, D)), _full((D, D)), _full((1, D))],
        out_specs=_rows((_BE, D)),
        out_shape=jax.ShapeDtypeStruct((E, D), jnp.float32),
    )(gs, gd, e, w1e, b1, w2, b2)


def _node_mlp_body(x_ref, agg_ref, w1x_ref, w1a_ref, b1_ref, w2_ref, b2_ref,
                   out_ref):
    t = (jnp.dot(x_ref[...], w1x_ref[...], preferred_element_type=jnp.float32)
         + jnp.dot(agg_ref[...], w1a_ref[...],
                   preferred_element_type=jnp.float32)
         + b1_ref[...])
    h = jnp.maximum(t, 0.0)
    out_ref[...] = (jnp.dot(h, w2_ref[...],
                            preferred_element_type=jnp.float32)
                    + b2_ref[...] + x_ref[...])


def _node_mlp(x, agg, w1x, w1a, b1, w2, b2):
    return pl.pallas_call(
        _node_mlp_body,
        grid=(N // _BN,),
        in_specs=[_rows((_BN, D)), _rows((_BN, D)),
                  _full((D, D)), _full((D, D)), _full((1, D)),
                  _full((D, D)), _full((1, D))],
        out_specs=_rows((_BN, D)),
        out_shape=jax.ShapeDtypeStruct((N, D), jnp.float32),
    )(x, agg, w1x, w1a, b1, w2, b2)


def _out_mlp_body(x_ref, w1_ref, b1_ref, w2_ref, b2_ref, w3_ref, b3_ref,
                  out_ref):
    h = jnp.maximum(jnp.dot(x_ref[...], w1_ref[...],
                            preferred_element_type=jnp.float32) + b1_ref[...],
                    0.0)
    h = jnp.maximum(jnp.dot(h, w2_ref[...],
                            preferred_element_type=jnp.float32) + b2_ref[...],
                    0.0)
    out_ref[...] = jnp.dot(h, w3_ref[...],
                           preferred_element_type=jnp.float32) + b3_ref[...]


def _out_mlp(x, w1, b1, w2, b2, w3p, b3p):
    return pl.pallas_call(
        _out_mlp_body,
        grid=(N // _BN,),
        in_specs=[_rows((_BN, D)),
                  _full((D, D)), _full((1, D)),
                  _full((D, D)), _full((1, D)),
                  _full((D, 128)), _full((1, 128))],
        out_specs=_rows((_BN, 128)),
        out_shape=jax.ShapeDtypeStruct((N, 128), jnp.float32),
    )(x, w1, b1, w2, b2, w3p, b3p)


_GC = 200
_EPW = E // NW

_mesh = plsc.VectorSubcoreMesh(core_axis_name="c", subcore_axis_name="s")


@functools.partial(
    pl.kernel,
    out_type=[jax.ShapeDtypeStruct((E, D), jnp.float32),
              jax.ShapeDtypeStruct((E, D), jnp.float32)],
    mesh=_mesh,
    scratch_types=[
        pltpu.VMEM((_GC,), jnp.int32),
        pltpu.VMEM((_GC,), jnp.int32),
        pltpu.VMEM((_GC, D), jnp.float32),
        pltpu.VMEM((_GC, D), jnp.float32),
        pltpu.SemaphoreType.DMA,
        pltpu.SemaphoreType.DMA,
    ],
)
def _sc_gather2(p_hbm, src_hbm, dst_hbm, gs_hbm, gd_hbm,
                si_v, di_v, rs_v, rd_v, sem_s, sem_d):
    wid = lax.axis_index("s") * NC + lax.axis_index("c")
    base = wid * _EPW

    def chunk(i, carry):
        off = base + i * _GC
        pltpu.sync_copy(src_hbm.at[pl.ds(off, _GC)], si_v)
        pltpu.sync_copy(dst_hbm.at[pl.ds(off, _GC)], di_v)
        cs = pltpu.async_copy(p_hbm.at[si_v], rs_v, sem_s)
        cd = pltpu.async_copy(p_hbm.at[di_v], rd_v, sem_d)
        cs.wait()
        pltpu.sync_copy(rs_v, gs_hbm.at[pl.ds(off, _GC)])
        cd.wait()
        pltpu.sync_copy(rd_v, gd_hbm.at[pl.ds(off, _GC)])
        return carry

    lax.fori_loop(0, _EPW // _GC, chunk, 0)


_SC_NPC = N // NS
_SCC = 200
_EPS = E // NS
_HD = D // NC


@functools.partial(
    pl.kernel,
    out_type=jax.ShapeDtypeStruct((N, D), jnp.float32),
    mesh=_mesh,
    scratch_types=[
        pltpu.VMEM((_SCC,), jnp.int32),
        pltpu.VMEM((_SCC, _HD), jnp.float32),
        pltpu.VMEM_SHARED((N, _HD), jnp.float32),
    ],
)
def _sc_scatter_add(enew_hbm, dst_hbm, zero_hbm, agg_hbm,
                    di_v, rows_v, acc_sh):
    c = lax.axis_index("c")
    s = lax.axis_index("s")
    pltpu.sync_copy(zero_hbm, acc_sh.at[pl.ds(s * _SC_NPC, _SC_NPC)])
    plsc.subcore_barrier()

    col = c * _HD
    base = s * _EPS

    def chunk(i, carry):
        off = base + i * _SCC
        pltpu.sync_copy(dst_hbm.at[pl.ds(off, _SCC)], di_v)
        pltpu.sync_copy(enew_hbm.at[pl.ds(off, _SCC), pl.ds(col, _HD)],
                        rows_v)
        pltpu.sync_copy(rows_v, acc_sh.at[di_v], add=True)
        return carry

    lax.fori_loop(0, _EPS // _SCC, chunk, 0)
    plsc.subcore_barrier()
    pltpu.sync_copy(acc_sh.at[pl.ds(s * _SC_NPC, _SC_NPC)],
                    agg_hbm.at[pl.ds(s * _SC_NPC, _SC_NPC), pl.ds(col, _HD)])


def kernel(x, edge_index, edge_attr, params):
    src = edge_index[0].astype(jnp.int32)
    dst = edge_index[1].astype(jnp.int32)
    dstN = dst + N
    zero = jnp.zeros((_SC_NPC, _HD), jnp.float32)

    for c in range(len(params['convs'])):
        ep = params['convs'][c]['edge_mlp']
        npar = params['convs'][c]['node_mlp']
        (w1, b1), (w2, b2) = ep
        (wn1, bn1), (wn2, bn2) = npar
        w1sd = jnp.concatenate([w1[:D], w1[D:2 * D]], axis=1)
        p = _node_proj(x, w1sd).reshape(2 * N, D)
        gs, gd = _sc_gather2(p, src, dstN)
        e_new = _edge_mlp(gs, gd, edge_attr, w1[2 * D:], b1.reshape(1, D),
                          w2, b2.reshape(1, D))
        agg = _sc_scatter_add(e_new, dst, zero)
        x = _node_mlp(x, agg, wn1[:D], wn1[D:], bn1.reshape(1, D),
                      wn2, bn2.reshape(1, D))
        edge_attr = e_new

    (wo1, bo1), (wo2, bo2), (wo3, bo3) = params['out']
    out_dim = wo3.shape[1]
    w3p = jnp.pad(wo3, ((0, 0), (0, 128 - out_dim)))
    b3p = jnp.pad(bo3, ((0, 128 - out_dim),))
    o = _out_mlp(x, wo1, bo1.reshape(1, D), wo2, bo2.reshape(1, D),
                 w3p, b3p.reshape(1, 128))
    return o[:, :out_dim]


# speedup vs baseline: 2.6226x; 2.6226x over previous
import functools

import jax
import jax.numpy as jnp
from jax import lax
from jax.experimental import pallas as pl
from jax.experimental.pallas import tpu as pltpu
from jax.experimental.pallas import tpu_sc as plsc

N = 10000
E = 160000
D = 256

NC = 2
NS = 16
NW = NC * NS


_BN = 2000
_BE = 2000


def _full(shape):
    return pl.BlockSpec(shape, lambda i: (0,) * len(shape))


def _rows(shape):
    return pl.BlockSpec(shape, lambda i: (i,) + (0,) * (len(shape) - 1))


def _node_proj_body(x_ref, w_ref, out_ref):
    d = jnp.dot(x_ref[...], w_ref[...], preferred_element_type=jnp.float32)
    out_ref[0] = d[:, :D]
    out_ref[1] = d[:, D:]


def _node_proj(x, w):
    return pl.pallas_call(
        _node_proj_body,
        grid=(N // _BN,),
        in_specs=[_rows((_BN, D)), _full((D, 2 * D))],
        out_specs=pl.BlockSpec((2, _BN, D), lambda i: (0, i, 0)),
        out_shape=jax.ShapeDtypeStruct((2, N, D), jnp.float32),
    )(x, w)


def _edge_mlp_body(gs_ref, gd_ref, e_ref, w1e_ref, b1_ref, w2_ref, b2_ref,
                   out_ref):
    t = (gs_ref[...] + gd_ref[...] + b1_ref[...]
         + jnp.dot(e_ref[...], w1e_ref[...],
                   preferred_element_type=jnp.float32))
    h = jnp.maximum(t, 0.0)
    out_ref[...] = (jnp.dot(h, w2_ref[...],
                            preferred_element_type=jnp.float32)
                    + b2_ref[...] + e_ref[...])


def _edge_mlp(gs, gd, e, w1e, b1, w2, b2):
    return pl.pallas_call(
        _edge_mlp_body,
        grid=(E // _BE,),
        in_specs=[_rows((_BE, D)), _rows((_BE, D)), _rows((_BE, D)),
                  _full((D, D)), _full((1, D)), _full((D, D)), _full((1, D))],
        out_specs=_rows((_BE, D)),
        out_shape=jax.ShapeDtypeStruct((E, D), jnp.float32),
    )(gs, gd, e, w1e, b1, w2, b2)


def _node_mlp_body(x_ref, agg_ref, w1x_ref, w1a_ref, b1_ref, w2_ref, b2_ref,
                   out_ref):
    t = (jnp.dot(x_ref[...], w1x_ref[...], preferred_element_type=jnp.float32)
         + jnp.dot(agg_ref[...], w1a_ref[...],
                   preferred_element_type=jnp.float32)
         + b1_ref[...])
    h = jnp.maximum(t, 0.0)
    out_ref[...] = (jnp.dot(h, w2_ref[...],
                            preferred_element_type=jnp.float32)
                    + b2_ref[...] + x_ref[...])


def _node_mlp(x, agg, w1x, w1a, b1, w2, b2):
    return pl.pallas_call(
        _node_mlp_body,
        grid=(N // _BN,),
        in_specs=[_rows((_BN, D)), _rows((_BN, D)),
                  _full((D, D)), _full((D, D)), _full((1, D)),
                  _full((D, D)), _full((1, D))],
        out_specs=_rows((_BN, D)),
        out_shape=jax.ShapeDtypeStruct((N, D), jnp.float32),
    )(x, agg, w1x, w1a, b1, w2, b2)


def _out_mlp_body(x_ref, w1_ref, b1_ref, w2_ref, b2_ref, w3_ref, b3_ref,
                  out_ref):
    h = jnp.maximum(jnp.dot(x_ref[...], w1_ref[...],
                            preferred_element_type=jnp.float32) + b1_ref[...],
                    0.0)
    h = jnp.maximum(jnp.dot(h, w2_ref[...],
                            preferred_element_type=jnp.float32) + b2_ref[...],
                    0.0)
    out_ref[...] = jnp.dot(h, w3_ref[...],
                           preferred_element_type=jnp.float32) + b3_ref[...]


def _out_mlp(x, w1, b1, w2, b2, w3p, b3p):
    return pl.pallas_call(
        _out_mlp_body,
        grid=(N // _BN,),
        in_specs=[_rows((_BN, D)),
                  _full((D, D)), _full((1, D)),
                  _full((D, D)), _full((1, D)),
                  _full((D, 128)), _full((1, 128))],
        out_specs=_rows((_BN, 128)),
        out_shape=jax.ShapeDtypeStruct((N, 128), jnp.float32),
    )(x, w1, b1, w2, b2, w3p, b3p)


_GC = 200
_EPW = E // NW


@functools.lru_cache(maxsize=None)
def _make_sc_gather2():
    mesh = plsc.VectorSubcoreMesh(core_axis_name="c", subcore_axis_name="s")

    @functools.partial(
        pl.kernel,
        out_type=[jax.ShapeDtypeStruct((E, D), jnp.float32),
                  jax.ShapeDtypeStruct((E, D), jnp.float32)],
        mesh=mesh,
        scratch_types=[
            pltpu.VMEM((_GC,), jnp.int32),
            pltpu.VMEM((_GC,), jnp.int32),
            pltpu.VMEM((_GC, D), jnp.float32),
            pltpu.VMEM((_GC, D), jnp.float32),
            pltpu.SemaphoreType.DMA,
            pltpu.SemaphoreType.DMA,
        ],
    )
    def sc_gather2(p_hbm, src_hbm, dst_hbm, gs_hbm, gd_hbm,
                   si_v, di_v, rs_v, rd_v, sem_s, sem_d):
        wid = lax.axis_index("s") * NC + lax.axis_index("c")
        base = wid * _EPW

        def chunk(i, carry):
            off = base + i * _GC
            pltpu.sync_copy(src_hbm.at[pl.ds(off, _GC)], si_v)
            pltpu.sync_copy(dst_hbm.at[pl.ds(off, _GC)], di_v)
            cs = pltpu.async_copy(p_hbm.at[si_v], rs_v, sem_s)
            cd = pltpu.async_copy(p_hbm.at[di_v], rd_v, sem_d)
            cs.wait()
            pltpu.sync_copy(rs_v, gs_hbm.at[pl.ds(off, _GC)])
            cd.wait()
            pltpu.sync_copy(rd_v, gd_hbm.at[pl.ds(off, _GC)])
            return carry

        lax.fori_loop(0, _EPW // _GC, chunk, 0)

    return sc_gather2


def _sc_gather2(p, src, dstN):
    return _make_sc_gather2()(p, src, dstN)


_NP = 10240
_SC_NPC = _NP // NS
_SCC = 200
_EPS = E // NS
_HD = D // NC


@functools.lru_cache(maxsize=None)
def _make_sc_scatter_add():
    mesh = plsc.VectorSubcoreMesh(core_axis_name="c", subcore_axis_name="s")

    @functools.partial(
        pl.kernel,
        out_type=jax.ShapeDtypeStruct((_NP, D), jnp.float32),
        mesh=mesh,
        scratch_types=[
            pltpu.VMEM((_SCC,), jnp.int32),
            pltpu.VMEM((_SCC, _HD), jnp.float32),
            pltpu.VMEM_SHARED((_NP, _HD), jnp.float32),
        ],
    )
    def sc_scatter_add(enew_hbm, dst_hbm, zero_hbm, agg_hbm,
                       di_v, rows_v, acc_sh):
        c = lax.axis_index("c")
        s = lax.axis_index("s")
        pltpu.sync_copy(zero_hbm, acc_sh.at[pl.ds(s * _SC_NPC, _SC_NPC)])
        plsc.subcore_barrier()

        col = c * _HD
        base = s * _EPS

        def chunk(i, carry):
            off = base + i * _SCC
            pltpu.sync_copy(dst_hbm.at[pl.ds(off, _SCC)], di_v)
            pltpu.sync_copy(enew_hbm.at[pl.ds(off, _SCC), pl.ds(col, _HD)],
                            rows_v)
            pltpu.sync_copy(rows_v, acc_sh.at[di_v], add=True)
            return carry

        lax.fori_loop(0, _EPS // _SCC, chunk, 0)
        plsc.subcore_barrier()
        pltpu.sync_copy(
            acc_sh.at[pl.ds(s * _SC_NPC, _SC_NPC)],
            agg_hbm.at[pl.ds(s * _SC_NPC, _SC_NPC), pl.ds(col, _HD)])

    return sc_scatter_add


def _sc_scatter_add(e_new, dst, zero):
    return _make_sc_scatter_add()(e_new, dst, zero)[:N]


def kernel(x, edge_index, edge_attr, params):
    src = edge_index[0].astype(jnp.int32)
    dst = edge_index[1].astype(jnp.int32)
    dstN = dst + N
    zero = jnp.zeros((_SC_NPC, _HD), jnp.float32)

    for c in range(len(params['convs'])):
        ep = params['convs'][c]['edge_mlp']
        npar = params['convs'][c]['node_mlp']
        (w1, b1), (w2, b2) = ep
        (wn1, bn1), (wn2, bn2) = npar
        w1sd = jnp.concatenate([w1[:D], w1[D:2 * D]], axis=1)
        p = _node_proj(x, w1sd).reshape(2 * N, D)
        gs, gd = _sc_gather2(p, src, dstN)
        e_new = _edge_mlp(gs, gd, edge_attr, w1[2 * D:], b1.reshape(1, D),
                          w2, b2.reshape(1, D))
        agg = _sc_scatter_add(e_new, dst, zero)
        x = _node_mlp(x, agg, wn1[:D], wn1[D:], bn1.reshape(1, D),
                      wn2, bn2.reshape(1, D))
        edge_attr = e_new

    (wo1, bo1), (wo2, bo2), (wo3, bo3) = params['out']
    out_dim = wo3.shape[1]
    w3p = jnp.pad(wo3, ((0, 0), (0, 128 - out_dim)))
    b3p = jnp.pad(bo3, ((0, 128 - out_dim),))
    o = _out_mlp(x, wo1, bo1.reshape(1, D), wo2, bo2.reshape(1, D),
                 w3p, b3p.reshape(1, 128))
    return o[:, :out_dim]
